# baseline (speedup 1.0000x reference)
import jax
import jax.numpy as jnp
from jax.experimental import pallas as pl
from jax.experimental.pallas import tpu as pltpu

N_Q, N_X, N_Z = 80, 24, 24
N = N_Q + N_X + N_Z
B = 256
HID = 256
EPS = 1e-5
G = 32
STEPS = B // G
NROWS = B * N

_BF16 = jnp.bfloat16
_F32 = jnp.float32


def _adj_prep(adj_blk):
    a = (adj_blk != 0).astype(_F32)
    row = jax.lax.broadcasted_iota(jnp.int32, (N, N), 0)
    col = jax.lax.broadcasted_iota(jnp.int32, (N, N), 1)
    a_hat = jnp.where((row == col)[None, :, :], 1.0, a)
    deg = jnp.sum(a_hat, axis=1)
    dinv = 1.0 / jnp.sqrt(deg)
    return a_hat, dinv


def _agg_raw(ah16, dinv, lin):
    msg = (dinv[:, :, None] * lin).astype(_BF16)
    mm = jax.lax.dot_general(
        ah16, msg, (((1,), (1,)), ((0,), (0,))),
        preferred_element_type=_F32)
    return mm * dinv[:, :, None]


def _matmul(h, w_ref):
    gg = h.shape[0]
    return jax.lax.dot_general(
        h.reshape(gg * N, HID).astype(_BF16), w_ref[...].astype(_BF16),
        (((1,), (0,)), ((), ())),
        preferred_element_type=_F32).reshape(gg, N, HID)


def _acc_stats(i, raw, st_scr, r):
    r16 = raw.reshape(G * N, HID).astype(_BF16)
    ones = (jax.lax.broadcasted_iota(jnp.int32, (1, G * N), 1) >= 0) \
        .astype(_BF16)
    s = jax.lax.dot_general(ones, r16, (((1,), (0,)), ((), ())),
                            preferred_element_type=_F32)
    q = jax.lax.dot_general(ones, r16 * r16, (((1,), (0,)), ((), ())),
                            preferred_element_type=_F32)
    @pl.when(i == 0)
    def _():
        st_scr[r:r + 2, :] = jnp.zeros((2, HID), _F32)
    st_scr[r:r + 2, :] += jnp.concatenate([s, q], axis=0)


def _bn_coeffs(st_scr, r, bias, g, bt):
    sm = st_scr[r:r + 1, :] * (1.0 / NROWS)
    sq = st_scr[r + 1:r + 2, :] * (1.0 / NROWS)
    sv = jax.lax.rsqrt((sq - sm * sm) + EPS) * g
    cv = bt - sm * sv
    return sv, cv


def _lin0(gam_ref, bet_ref, w0_ref):
    ridx = jax.lax.broadcasted_iota(jnp.int32, (N, 1), 0)
    group = jnp.where(ridx < N_Q, 0, jnp.where(ridx < N_Q + N_X, 1, 2))
    cidx = jax.lax.broadcasted_iota(jnp.int32, (N, 3), 1)
    x = (cidx == group).astype(_F32)
    c3 = jax.lax.broadcasted_iota(jnp.int32, (1, 3), 1)
    p = jnp.where(c3 == 0, N_Q / N,
                  jnp.where(c3 == 1, N_X / N, N_Z / N)).astype(_F32)
    xf = (x - p) / jnp.sqrt(p * (1.0 - p) + EPS) * gam_ref[...] + bet_ref[...]
    return jax.lax.dot_general(
        xf.astype(_BF16), w0_ref[...].astype(_BF16), (((1,), (0,)), ((), ())),
        preferred_element_type=_F32)


def _fused(adj_ref, gam_ref, bet_ref, w0_ref, b0_ref, g0_ref, bt0_ref,
           w1_ref, b1_ref, g1_ref, bt1_ref, w2_ref, b2_ref, g2_ref, bt2_ref,
           lng_ref, lnb_ref, out_ref,
           ah_scr, dinv_scr, h1_scr, h2_scr, st_scr):
    s = pl.program_id(0)
    i = pl.program_id(1)
    sl = pl.ds(i * G, G)

    @pl.when(s == 0)
    def _():
        a_hat, dinv = _adj_prep(adj_ref[...])
        ah16 = a_hat.astype(_BF16)
        ah_scr[sl] = ah16
        dinv_scr[sl] = dinv
        lin = jnp.broadcast_to(
            _lin0(gam_ref, bet_ref, w0_ref)[None], (G, N, HID))
        raw0 = _agg_raw(ah16, dinv, lin)
        _acc_stats(i, raw0, st_scr, 0)

    @pl.when(s == 1)
    def _():
        ah16 = ah_scr[sl]
        dinv = dinv_scr[sl]
        lin = jnp.broadcast_to(
            _lin0(gam_ref, bet_ref, w0_ref)[None], (G, N, HID))
        raw0 = _agg_raw(ah16, dinv, lin)
        sv, cv = _bn_coeffs(st_scr, 0, b0_ref[...], g0_ref[...], bt0_ref[...])
        h1 = jnp.maximum(raw0 * sv[None] + cv[None], 0.0)
        h1_scr[sl] = h1.astype(_BF16)
        raw1 = _agg_raw(ah16, dinv, _matmul(h1, w1_ref))
        _acc_stats(i, raw1, st_scr, 2)

    @pl.when(s == 2)
    def _():
        ah16 = ah_scr[sl]
        dinv = dinv_scr[sl]
        h1 = h1_scr[sl].astype(_F32)
        raw1 = _agg_raw(ah16, dinv, _matmul(h1, w1_ref))
        sv, cv = _bn_coeffs(st_scr, 2, b1_ref[...], g1_ref[...], bt1_ref[...])
        h2 = jnp.maximum(raw1 * sv[None] + cv[None] + h1, 0.0)
        h2_scr[sl] = h2.astype(_BF16)
        raw2 = _agg_raw(ah16, dinv, _matmul(h2, w2_ref))
        _acc_stats(i, raw2, st_scr, 4)

    @pl.when(s == 3)
    def _():
        ah16 = ah_scr[sl]
        dinv = dinv_scr[sl]
        h2 = h2_scr[sl].astype(_F32)
        raw2 = _agg_raw(ah16, dinv, _matmul(h2, w2_ref))
        sv, cv = _bn_coeffs(st_scr, 4, b2_ref[...], g2_ref[...], bt2_ref[...])
        h3 = jnp.maximum(raw2 * sv[None] + cv[None] + h2, 0.0)
        pooled = jnp.mean(h3, axis=1)
        mu = jnp.mean(pooled, axis=-1, keepdims=True)
        var = jnp.mean(pooled * pooled, axis=-1, keepdims=True) - mu * mu
        out_ref[...] = (pooled - mu) * jax.lax.rsqrt(var + EPS) \
            * lng_ref[...] + lnb_ref[...]


_FIX2 = lambda s, i: (0, 0)
_ROW_SPEC = pl.BlockSpec((1, HID), _FIX2)
_W_SPEC = pl.BlockSpec((HID, HID), _FIX2)


def kernel(adjacency_matrices, in_gamma, in_beta, W0, b0, bn_g0, bn_b0,
           W1, b1, bn_g1, bn_b1, W2, b2, bn_g2, bn_b2, ln_g, ln_b):
    adj = adjacency_matrices.astype(jnp.int8)
    gamc = in_gamma.reshape(1, 3)
    betc = in_beta.reshape(1, 3)
    row = lambda v: v.reshape(1, HID)
    g3spec = pl.BlockSpec((1, 3), _FIX2)
    w0spec = pl.BlockSpec((3, HID), _FIX2)
    adj_spec = pl.BlockSpec(
        (G, N, N), lambda s, i: (jnp.where(s == 0, i, 0), 0, 0))

    out = pl.pallas_call(
        _fused,
        grid=(4, STEPS),
        in_specs=[adj_spec, g3spec, g3spec, w0spec, _ROW_SPEC,
                  _ROW_SPEC, _ROW_SPEC, _W_SPEC, _ROW_SPEC,
                  _ROW_SPEC, _ROW_SPEC, _W_SPEC, _ROW_SPEC,
                  _ROW_SPEC, _ROW_SPEC, _ROW_SPEC, _ROW_SPEC],
        out_specs=pl.BlockSpec((G, HID), lambda s, i: (i, 0)),
        out_shape=jax.ShapeDtypeStruct((B, HID), _F32),
        scratch_shapes=[
            pltpu.VMEM((B, N, N), _BF16),
            pltpu.VMEM((B, N), _F32),
            pltpu.VMEM((B, N, HID), _BF16),
            pltpu.VMEM((B, N, HID), _BF16),
            pltpu.VMEM((8, HID), _F32),
        ],
    )(adj, gamc, betc, W0, row(b0), row(bn_g0), row(bn_b0),
      W1, row(b1), row(bn_g1), row(bn_b1),
      W2, row(b2), row(bn_g2), row(bn_b2), row(ln_g), row(ln_b))

    return out


# speedup vs baseline: 1.0308x; 1.0308x over previous
import jax
import jax.numpy as jnp
from jax.experimental import pallas as pl
from jax.experimental.pallas import tpu as pltpu

N_Q, N_X, N_Z = 80, 24, 24
N = N_Q + N_X + N_Z
B = 256
HID = 256
EPS = 1e-5
G = 32
STEPS = B // G
NROWS = B * N

_BF16 = jnp.bfloat16
_F32 = jnp.float32


def _adj_prep(adj_blk):
    a = (adj_blk != 0).astype(_F32)
    row = jax.lax.broadcasted_iota(jnp.int32, (N, N), 0)
    col = jax.lax.broadcasted_iota(jnp.int32, (N, N), 1)
    a_hat = jnp.where((row == col)[None, :, :], 1.0, a)
    deg = jnp.sum(a_hat, axis=1)
    dinv = 1.0 / jnp.sqrt(deg)
    return a_hat, dinv


def _agg_raw(ah16, dinv, lin):
    msg = (dinv[:, :, None] * lin).astype(_BF16)
    mm = jax.lax.dot_general(
        ah16, msg, (((1,), (1,)), ((0,), (0,))),
        preferred_element_type=_F32)
    return mm * dinv[:, :, None]


def _matmul(h, w_ref):
    gg = h.shape[0]
    return jax.lax.dot_general(
        h.reshape(gg * N, HID).astype(_BF16), w_ref[...].astype(_BF16),
        (((1,), (0,)), ((), ())),
        preferred_element_type=_F32).reshape(gg, N, HID)


def _acc_stats(i, raw, st_scr, r):
    s = jnp.sum(raw, axis=(0, 1))
    q = jnp.sum(raw * raw, axis=(0, 1))
    @pl.when(i == 0)
    def _():
        st_scr[r:r + 2, :] = jnp.zeros((2, HID), _F32)
    st_scr[r:r + 2, :] += jnp.stack([s, q], axis=0)


def _bn_coeffs(st_scr, r, bias, g, bt):
    sm = st_scr[r:r + 1, :] * (1.0 / NROWS)
    sq = st_scr[r + 1:r + 2, :] * (1.0 / NROWS)
    sv = jax.lax.rsqrt((sq - sm * sm) + EPS) * g
    cv = bt - sm * sv
    return sv, cv


def _lin0(gam_ref, bet_ref, w0_ref):
    ridx = jax.lax.broadcasted_iota(jnp.int32, (N, 1), 0)
    group = jnp.where(ridx < N_Q, 0, jnp.where(ridx < N_Q + N_X, 1, 2))
    cidx = jax.lax.broadcasted_iota(jnp.int32, (N, 3), 1)
    x = (cidx == group).astype(_F32)
    c3 = jax.lax.broadcasted_iota(jnp.int32, (1, 3), 1)
    p = jnp.where(c3 == 0, N_Q / N,
                  jnp.where(c3 == 1, N_X / N, N_Z / N)).astype(_F32)
    xf = (x - p) / jnp.sqrt(p * (1.0 - p) + EPS) * gam_ref[...] + bet_ref[...]
    return jax.lax.dot_general(
        xf.astype(_BF16), w0_ref[...].astype(_BF16), (((1,), (0,)), ((), ())),
        preferred_element_type=_F32)


def _fused(adj_ref, gam_ref, bet_ref, w0_ref, b0_ref, g0_ref, bt0_ref,
           w1_ref, b1_ref, g1_ref, bt1_ref, w2_ref, b2_ref, g2_ref, bt2_ref,
           lng_ref, lnb_ref, out_ref,
           ah_scr, dinv_scr, h1_scr, h2_scr, st_scr):
    s = pl.program_id(0)
    i = pl.program_id(1)
    sl = pl.ds(i * G, G)

    @pl.when(s == 0)
    def _():
        a_hat, dinv = _adj_prep(adj_ref[...])
        ah16 = a_hat.astype(_BF16)
        ah_scr[sl] = ah16
        dinv_scr[sl] = dinv
        lin = jnp.broadcast_to(
            _lin0(gam_ref, bet_ref, w0_ref)[None], (G, N, HID))
        raw0 = _agg_raw(ah16, dinv, lin)
        _acc_stats(i, raw0, st_scr, 0)

    @pl.when(s == 1)
    def _():
        ah16 = ah_scr[sl]
        dinv = dinv_scr[sl]
        lin = jnp.broadcast_to(
            _lin0(gam_ref, bet_ref, w0_ref)[None], (G, N, HID))
        raw0 = _agg_raw(ah16, dinv, lin)
        sv, cv = _bn_coeffs(st_scr, 0, b0_ref[...], g0_ref[...], bt0_ref[...])
        h1 = jnp.maximum(raw0 * sv[None] + cv[None], 0.0)
        h1_scr[sl] = h1.astype(_BF16)
        raw1 = _agg_raw(ah16, dinv, _matmul(h1, w1_ref))
        _acc_stats(i, raw1, st_scr, 2)

    @pl.when(s == 2)
    def _():
        ah16 = ah_scr[sl]
        dinv = dinv_scr[sl]
        h1 = h1_scr[sl].astype(_F32)
        raw1 = _agg_raw(ah16, dinv, _matmul(h1, w1_ref))
        sv, cv = _bn_coeffs(st_scr, 2, b1_ref[...], g1_ref[...], bt1_ref[...])
        h2 = jnp.maximum(raw1 * sv[None] + cv[None] + h1, 0.0)
        h2_scr[sl] = h2.astype(_BF16)
        raw2 = _agg_raw(ah16, dinv, _matmul(h2, w2_ref))
        _acc_stats(i, raw2, st_scr, 4)

    @pl.when(s == 3)
    def _():
        ah16 = ah_scr[sl]
        dinv = dinv_scr[sl]
        h2 = h2_scr[sl].astype(_F32)
        raw2 = _agg_raw(ah16, dinv, _matmul(h2, w2_ref))
        sv, cv = _bn_coeffs(st_scr, 4, b2_ref[...], g2_ref[...], bt2_ref[...])
        h3 = jnp.maximum(raw2 * sv[None] + cv[None] + h2, 0.0)
        pooled = jnp.mean(h3, axis=1)
        mu = jnp.mean(pooled, axis=-1, keepdims=True)
        var = jnp.mean(pooled * pooled, axis=-1, keepdims=True) - mu * mu
        out_ref[...] = (pooled - mu) * jax.lax.rsqrt(var + EPS) \
            * lng_ref[...] + lnb_ref[...]


_FIX2 = lambda s, i: (0, 0)
_ROW_SPEC = pl.BlockSpec((1, HID), _FIX2)
_W_SPEC = pl.BlockSpec((HID, HID), _FIX2)


def kernel(adjacency_matrices, in_gamma, in_beta, W0, b0, bn_g0, bn_b0,
           W1, b1, bn_g1, bn_b1, W2, b2, bn_g2, bn_b2, ln_g, ln_b):
    adj = adjacency_matrices.astype(jnp.int8)
    gamc = in_gamma.reshape(1, 3)
    betc = in_beta.reshape(1, 3)
    row = lambda v: v.reshape(1, HID)
    g3spec = pl.BlockSpec((1, 3), _FIX2)
    w0spec = pl.BlockSpec((3, HID), _FIX2)
    adj_spec = pl.BlockSpec(
        (G, N, N), lambda s, i: (jnp.where(s == 0, i, 0), 0, 0))

    out = pl.pallas_call(
        _fused,
        grid=(4, STEPS),
        in_specs=[adj_spec, g3spec, g3spec, w0spec, _ROW_SPEC,
                  _ROW_SPEC, _ROW_SPEC, _W_SPEC, _ROW_SPEC,
                  _ROW_SPEC, _ROW_SPEC, _W_SPEC, _ROW_SPEC,
                  _ROW_SPEC, _ROW_SPEC, _ROW_SPEC, _ROW_SPEC],
        out_specs=pl.BlockSpec((G, HID), lambda s, i: (i, 0)),
        out_shape=jax.ShapeDtypeStruct((B, HID), _F32),
        scratch_shapes=[
            pltpu.VMEM((B, N, N), _BF16),
            pltpu.VMEM((B, N), _F32),
            pltpu.VMEM((B, N, HID), _BF16),
            pltpu.VMEM((B, N, HID), _BF16),
            pltpu.VMEM((8, HID), _F32),
        ],
    )(adj, gamc, betc, W0, row(b0), row(bn_g0), row(bn_b0),
      W1, row(b1), row(bn_g1), row(bn_b1),
      W2, row(b2), row(bn_g2), row(bn_b2), row(ln_g), row(ln_b))

    return out


# speedup vs baseline: 1.1000x; 1.0671x over previous
import jax
import jax.numpy as jnp
from jax.experimental import pallas as pl
from jax.experimental.pallas import tpu as pltpu

N_Q, N_X, N_Z = 80, 24, 24
N = N_Q + N_X + N_Z
B = 256
HID = 256
EPS = 1e-5
G = 64
STEPS = B // G
NROWS = B * N

_BF16 = jnp.bfloat16
_F32 = jnp.float32


def _adj_prep(adj_blk):
    a = (adj_blk != 0).astype(_F32)
    row = jax.lax.broadcasted_iota(jnp.int32, (N, N), 0)
    col = jax.lax.broadcasted_iota(jnp.int32, (N, N), 1)
    a_hat = jnp.where((row == col)[None, :, :], 1.0, a)
    deg = jnp.sum(a_hat, axis=1)
    dinv = 1.0 / jnp.sqrt(deg)
    return a_hat, dinv


def _agg_raw(ah16, dinv, lin):
    msg = (dinv[:, :, None] * lin).astype(_BF16)
    mm = jax.lax.dot_general(
        ah16, msg, (((1,), (1,)), ((0,), (0,))),
        preferred_element_type=_F32)
    return mm * dinv[:, :, None]


def _matmul(h, w_ref):
    gg = h.shape[0]
    return jax.lax.dot_general(
        h.reshape(gg * N, HID).astype(_BF16), w_ref[...].astype(_BF16),
        (((1,), (0,)), ((), ())),
        preferred_element_type=_F32).reshape(gg, N, HID)


def _acc_stats(i, raw, st_scr, r):
    s = jnp.sum(raw, axis=(0, 1))
    q = jnp.sum(raw * raw, axis=(0, 1))
    @pl.when(i == 0)
    def _():
        st_scr[r:r + 2, :] = jnp.zeros((2, HID), _F32)
    st_scr[r:r + 2, :] += jnp.stack([s, q], axis=0)


def _bn_coeffs(st_scr, r, bias, g, bt):
    sm = st_scr[r:r + 1, :] * (1.0 / NROWS)
    sq = st_scr[r + 1:r + 2, :] * (1.0 / NROWS)
    sv = jax.lax.rsqrt((sq - sm * sm) + EPS) * g
    cv = bt - sm * sv
    return sv, cv


def _lin0(gam_ref, bet_ref, w0_ref):
    ridx = jax.lax.broadcasted_iota(jnp.int32, (N, 1), 0)
    group = jnp.where(ridx < N_Q, 0, jnp.where(ridx < N_Q + N_X, 1, 2))
    cidx = jax.lax.broadcasted_iota(jnp.int32, (N, 3), 1)
    x = (cidx == group).astype(_F32)
    c3 = jax.lax.broadcasted_iota(jnp.int32, (1, 3), 1)
    p = jnp.where(c3 == 0, N_Q / N,
                  jnp.where(c3 == 1, N_X / N, N_Z / N)).astype(_F32)
    xf = (x - p) / jnp.sqrt(p * (1.0 - p) + EPS) * gam_ref[...] + bet_ref[...]
    return jax.lax.dot_general(
        xf.astype(_BF16), w0_ref[...].astype(_BF16), (((1,), (0,)), ((), ())),
        preferred_element_type=_F32)


def _fused(adj_ref, gam_ref, bet_ref, w0_ref, b0_ref, g0_ref, bt0_ref,
           w1_ref, b1_ref, g1_ref, bt1_ref, w2_ref, b2_ref, g2_ref, bt2_ref,
           lng_ref, lnb_ref, out_ref,
           ah_scr, dinv_scr, h1_scr, h2_scr, st_scr):
    s = pl.program_id(0)
    i = pl.program_id(1)
    sl = pl.ds(i * G, G)

    @pl.when(s == 0)
    def _():
        a_hat, dinv = _adj_prep(adj_ref[...])
        ah16 = a_hat.astype(_BF16)
        ah_scr[sl] = ah16
        dinv_scr[sl] = dinv
        lin = jnp.broadcast_to(
            _lin0(gam_ref, bet_ref, w0_ref)[None], (G, N, HID))
        raw0 = _agg_raw(ah16, dinv, lin)
        _acc_stats(i, raw0, st_scr, 0)

    @pl.when(s == 1)
    def _():
        ah16 = ah_scr[sl]
        dinv = dinv_scr[sl]
        lin = jnp.broadcast_to(
            _lin0(gam_ref, bet_ref, w0_ref)[None], (G, N, HID))
        raw0 = _agg_raw(ah16, dinv, lin)
        sv, cv = _bn_coeffs(st_scr, 0, b0_ref[...], g0_ref[...], bt0_ref[...])
        h1 = jnp.maximum(raw0 * sv[None] + cv[None], 0.0)
        h1_scr[sl] = h1.astype(_BF16)
        raw1 = _agg_raw(ah16, dinv, _matmul(h1, w1_ref))
        _acc_stats(i, raw1, st_scr, 2)

    @pl.when(s == 2)
    def _():
        ah16 = ah_scr[sl]
        dinv = dinv_scr[sl]
        h1 = h1_scr[sl].astype(_F32)
        raw1 = _agg_raw(ah16, dinv, _matmul(h1, w1_ref))
        sv, cv = _bn_coeffs(st_scr, 2, b1_ref[...], g1_ref[...], bt1_ref[...])
        h2 = jnp.maximum(raw1 * sv[None] + cv[None] + h1, 0.0)
        h2_scr[sl] = h2.astype(_BF16)
        raw2 = _agg_raw(ah16, dinv, _matmul(h2, w2_ref))
        _acc_stats(i, raw2, st_scr, 4)

    @pl.when(s == 3)
    def _():
        ah16 = ah_scr[sl]
        dinv = dinv_scr[sl]
        h2 = h2_scr[sl].astype(_F32)
        raw2 = _agg_raw(ah16, dinv, _matmul(h2, w2_ref))
        sv, cv = _bn_coeffs(st_scr, 4, b2_ref[...], g2_ref[...], bt2_ref[...])
        h3 = jnp.maximum(raw2 * sv[None] + cv[None] + h2, 0.0)
        pooled = jnp.mean(h3, axis=1)
        mu = jnp.mean(pooled, axis=-1, keepdims=True)
        var = jnp.mean(pooled * pooled, axis=-1, keepdims=True) - mu * mu
        out_ref[...] = (pooled - mu) * jax.lax.rsqrt(var + EPS) \
            * lng_ref[...] + lnb_ref[...]


_FIX2 = lambda s, i: (0, 0)
_ROW_SPEC = pl.BlockSpec((1, HID), _FIX2)
_W_SPEC = pl.BlockSpec((HID, HID), _FIX2)


def kernel(adjacency_matrices, in_gamma, in_beta, W0, b0, bn_g0, bn_b0,
           W1, b1, bn_g1, bn_b1, W2, b2, bn_g2, bn_b2, ln_g, ln_b):
    adj = adjacency_matrices.astype(jnp.int8)
    gamc = in_gamma.reshape(1, 3)
    betc = in_beta.reshape(1, 3)
    row = lambda v: v.reshape(1, HID)
    g3spec = pl.BlockSpec((1, 3), _FIX2)
    w0spec = pl.BlockSpec((3, HID), _FIX2)
    adj_spec = pl.BlockSpec(
        (G, N, N), lambda s, i: (jnp.where(s == 0, i, 0), 0, 0))

    out = pl.pallas_call(
        _fused,
        grid=(4, STEPS),
        in_specs=[adj_spec, g3spec, g3spec, w0spec, _ROW_SPEC,
                  _ROW_SPEC, _ROW_SPEC, _W_SPEC, _ROW_SPEC,
                  _ROW_SPEC, _ROW_SPEC, _W_SPEC, _ROW_SPEC,
                  _ROW_SPEC, _ROW_SPEC, _ROW_SPEC, _ROW_SPEC],
        out_specs=pl.BlockSpec((G, HID), lambda s, i: (i, 0)),
        out_shape=jax.ShapeDtypeStruct((B, HID), _F32),
        scratch_shapes=[
            pltpu.VMEM((B, N, N), _BF16),
            pltpu.VMEM((B, N), _F32),
            pltpu.VMEM((B, N, HID), _BF16),
            pltpu.VMEM((B, N, HID), _BF16),
            pltpu.VMEM((8, HID), _F32),
        ],
    )(adj, gamc, betc, W0, row(b0), row(bn_g0), row(bn_b0),
      W1, row(b1), row(bn_g1), row(bn_b1),
      W2, row(b2), row(bn_g2), row(bn_b2), row(ln_g), row(ln_b))

    return out


# speedup vs baseline: 1.1007x; 1.0007x over previous
import jax
import jax.numpy as jnp
from jax.experimental import pallas as pl
from jax.experimental.pallas import tpu as pltpu

N_Q, N_X, N_Z = 80, 24, 24
N = N_Q + N_X + N_Z
B = 256
HID = 256
EPS = 1e-5
G = 64
STEPS = B // G
NROWS = B * N

_BF16 = jnp.bfloat16
_F32 = jnp.float32


def _adj_prep(adj_blk):
    a = (adj_blk != 0).astype(_F32)
    row = jax.lax.broadcasted_iota(jnp.int32, (N, N), 0)
    col = jax.lax.broadcasted_iota(jnp.int32, (N, N), 1)
    a_hat = jnp.where((row == col)[None, :, :], 1.0, a)
    deg = jnp.sum(a_hat, axis=1)
    dinv = 1.0 / jnp.sqrt(deg)
    return a_hat, dinv


def _agg_raw(ah16, dinv, lin):
    msg = (dinv[:, :, None] * lin).astype(_BF16)
    mm = jax.lax.dot_general(
        ah16, msg, (((1,), (1,)), ((0,), (0,))),
        preferred_element_type=_F32)
    return mm * dinv[:, :, None]


def _matmul(h16, w_ref):
    gg = h16.shape[0]
    return jax.lax.dot_general(
        h16.reshape(gg * N, HID), w_ref[...].astype(_BF16),
        (((1,), (0,)), ((), ())),
        preferred_element_type=_F32).reshape(gg, N, HID)


def _acc_stats(i, raw, st_scr, r):
    s = jnp.sum(raw, axis=(0, 1))
    q = jnp.sum(raw * raw, axis=(0, 1))
    @pl.when(i == 0)
    def _():
        st_scr[r:r + 2, :] = jnp.zeros((2, HID), _F32)
    st_scr[r:r + 2, :] += jnp.stack([s, q], axis=0)


def _bn_coeffs(st_scr, r, bias, g, bt):
    sm = st_scr[r:r + 1, :] * (1.0 / NROWS)
    sq = st_scr[r + 1:r + 2, :] * (1.0 / NROWS)
    sv = jax.lax.rsqrt((sq - sm * sm) + EPS) * g
    cv = bt - sm * sv
    return sv, cv


def _lin0(gam_ref, bet_ref, w0_ref):
    ridx = jax.lax.broadcasted_iota(jnp.int32, (N, 1), 0)
    group = jnp.where(ridx < N_Q, 0, jnp.where(ridx < N_Q + N_X, 1, 2))
    cidx = jax.lax.broadcasted_iota(jnp.int32, (N, 3), 1)
    x = (cidx == group).astype(_F32)
    c3 = jax.lax.broadcasted_iota(jnp.int32, (1, 3), 1)
    p = jnp.where(c3 == 0, N_Q / N,
                  jnp.where(c3 == 1, N_X / N, N_Z / N)).astype(_F32)
    xf = (x - p) / jnp.sqrt(p * (1.0 - p) + EPS) * gam_ref[...] + bet_ref[...]
    return jax.lax.dot_general(
        xf.astype(_BF16), w0_ref[...].astype(_BF16), (((1,), (0,)), ((), ())),
        preferred_element_type=_F32)


def _fused(adj_ref, gam_ref, bet_ref, w0_ref, b0_ref, g0_ref, bt0_ref,
           w1_ref, b1_ref, g1_ref, bt1_ref, w2_ref, b2_ref, g2_ref, bt2_ref,
           lng_ref, lnb_ref, out_ref,
           ah_scr, dinv_scr, h1_scr, h2_scr, st_scr):
    s = pl.program_id(0)
    i = pl.program_id(1)
    sl = pl.ds(i * G, G)

    @pl.when(s == 0)
    def _():
        a_hat, dinv = _adj_prep(adj_ref[...])
        ah16 = a_hat.astype(_BF16)
        ah_scr[sl] = ah16
        dinv_scr[sl] = dinv
        lin = jnp.broadcast_to(
            _lin0(gam_ref, bet_ref, w0_ref)[None], (G, N, HID))
        raw0 = _agg_raw(ah16, dinv, lin)
        _acc_stats(i, raw0, st_scr, 0)

    @pl.when(s == 1)
    def _():
        ah16 = ah_scr[sl]
        dinv = dinv_scr[sl]
        lin = jnp.broadcast_to(
            _lin0(gam_ref, bet_ref, w0_ref)[None], (G, N, HID))
        raw0 = _agg_raw(ah16, dinv, lin)
        sv, cv = _bn_coeffs(st_scr, 0, b0_ref[...], g0_ref[...], bt0_ref[...])
        h1 = jnp.maximum(raw0 * sv[None] + cv[None], 0.0)
        h116 = h1.astype(_BF16)
        h1_scr[sl] = h116
        raw1 = _agg_raw(ah16, dinv, _matmul(h116, w1_ref))
        _acc_stats(i, raw1, st_scr, 2)

    @pl.when(s == 2)
    def _():
        ah16 = ah_scr[sl]
        dinv = dinv_scr[sl]
        h116 = h1_scr[sl]
        raw1 = _agg_raw(ah16, dinv, _matmul(h116, w1_ref))
        sv, cv = _bn_coeffs(st_scr, 2, b1_ref[...], g1_ref[...], bt1_ref[...])
        h2 = jnp.maximum(raw1 * sv[None] + cv[None] + h116.astype(_F32), 0.0)
        h216 = h2.astype(_BF16)
        h2_scr[sl] = h216
        raw2 = _agg_raw(ah16, dinv, _matmul(h216, w2_ref))
        _acc_stats(i, raw2, st_scr, 4)

    @pl.when(s == 3)
    def _():
        ah16 = ah_scr[sl]
        dinv = dinv_scr[sl]
        h216 = h2_scr[sl]
        raw2 = _agg_raw(ah16, dinv, _matmul(h216, w2_ref))
        sv, cv = _bn_coeffs(st_scr, 4, b2_ref[...], g2_ref[...], bt2_ref[...])
        h3 = jnp.maximum(raw2 * sv[None] + cv[None] + h216.astype(_F32), 0.0)
        pooled = jnp.mean(h3, axis=1)
        mu = jnp.mean(pooled, axis=-1, keepdims=True)
        var = jnp.mean(pooled * pooled, axis=-1, keepdims=True) - mu * mu
        out_ref[...] = (pooled - mu) * jax.lax.rsqrt(var + EPS) \
            * lng_ref[...] + lnb_ref[...]


_FIX2 = lambda s, i: (0, 0)
_ROW_SPEC = pl.BlockSpec((1, HID), _FIX2)
_W_SPEC = pl.BlockSpec((HID, HID), _FIX2)


def kernel(adjacency_matrices, in_gamma, in_beta, W0, b0, bn_g0, bn_b0,
           W1, b1, bn_g1, bn_b1, W2, b2, bn_g2, bn_b2, ln_g, ln_b):
    adj = adjacency_matrices.astype(jnp.int8)
    gamc = in_gamma.reshape(1, 3)
    betc = in_beta.reshape(1, 3)
    row = lambda v: v.reshape(1, HID)
    g3spec = pl.BlockSpec((1, 3), _FIX2)
    w0spec = pl.BlockSpec((3, HID), _FIX2)
    adj_spec = pl.BlockSpec(
        (G, N, N), lambda s, i: (jnp.where(s == 0, i, 0), 0, 0))

    out = pl.pallas_call(
        _fused,
        grid=(4, STEPS),
        in_specs=[adj_spec, g3spec, g3spec, w0spec, _ROW_SPEC,
                  _ROW_SPEC, _ROW_SPEC, _W_SPEC, _ROW_SPEC,
                  _ROW_SPEC, _ROW_SPEC, _W_SPEC, _ROW_SPEC,
                  _ROW_SPEC, _ROW_SPEC, _ROW_SPEC, _ROW_SPEC],
        out_specs=pl.BlockSpec((G, HID), lambda s, i: (i, 0)),
        out_shape=jax.ShapeDtypeStruct((B, HID), _F32),
        scratch_shapes=[
            pltpu.VMEM((B, N, N), _BF16),
            pltpu.VMEM((B, N), _F32),
            pltpu.VMEM((B, N, HID), _BF16),
            pltpu.VMEM((B, N, HID), _BF16),
            pltpu.VMEM((8, HID), _F32),
        ],
    )(adj, gamc, betc, W0, row(b0), row(bn_g0), row(bn_b0),
      W1, row(b1), row(bn_g1), row(bn_b1),
      W2, row(b2), row(bn_g2), row(bn_b2), row(ln_g), row(ln_b))

    return out
